# Initial kernel scaffold; baseline (speedup 1.0000x reference)
import jax
import jax.numpy as jnp
from jax.experimental import pallas as pl


def kernel(hidden_states, Wg, Wgu, Wd):
    raise NotImplementedError("write your pallas kernel here")


# speedup vs baseline: 3.1496x; 3.1496x over previous
import functools

import jax
import jax.numpy as jnp
from jax import lax
from jax.experimental import pallas as pl
from jax.experimental.pallas import tpu as pltpu
from jax.experimental.pallas import tpu_sc as plsc

E = 16
D = 1024
F = 768
T = 4096
R = 256
G = (2 * T) // R + E
S = G * R
C = 512

NC = 2
NS = 16
NW = NC * NS
TOK_W = T // NW
CH = 32


def _router_body(x_ref, wg_ref, meta_ref, blk_ref, oh_scr, rank_scr):
    x = x_ref[...]
    wg = wg_ref[...]
    logits = lax.dot_general(x, wg, (((1,), (1,)), ((), ())),
                             preferred_element_type=jnp.float32)
    lane = lax.broadcasted_iota(jnp.int32, (T, E), 1)
    m1 = jnp.max(logits, axis=1, keepdims=True)
    e0 = jnp.min(jnp.where(logits == m1, lane, E), axis=1, keepdims=True)
    logits2 = jnp.where(lane == e0, -jnp.inf, logits)
    m2 = jnp.max(logits2, axis=1, keepdims=True)
    e1 = jnp.min(jnp.where(logits2 == m2, lane, E), axis=1, keepdims=True)
    w0 = 1.0 / (1.0 + jnp.exp(m2 - m1))
    w1 = 1.0 - w0

    oh0 = (lane == e0).astype(jnp.float32)
    oh1 = (lane == e1).astype(jnp.float32)
    oh_scr[pl.ds(0, T), :] = oh0
    oh_scr[pl.ds(T, T), :] = oh1

    tri = (lax.broadcasted_iota(jnp.int32, (C, C), 0)
           >= lax.broadcasted_iota(jnp.int32, (C, C), 1)).astype(jnp.float32)

    def chunk(c, carry):
        blkv = oh_scr[pl.ds(c * C, C), :]
        inc = lax.dot_general(tri, blkv, (((1,), (0,)), ((), ())),
                              preferred_element_type=jnp.float32)
        rank_scr[pl.ds(c * C, C), :] = inc + carry
        return carry + inc[C - 1:C, :]

    counts = lax.fori_loop(0, (2 * T) // C, chunk,
                           jnp.zeros((1, E), jnp.float32))

    nblk = jnp.ceil(counts / R)
    er = lax.broadcasted_iota(jnp.int32, (E, E), 0)
    ec = lax.broadcasted_iota(jnp.int32, (E, E), 1)
    m_lt = (er < ec).astype(jnp.float32)
    m_le = (er <= ec).astype(jnp.float32)
    pad_base = lax.dot_general(nblk, m_lt, (((1,), (0,)), ((), ())),
                               preferred_element_type=jnp.float32) * R
    cum_incl = lax.dot_general(nblk, m_le, (((1,), (0,)), ((), ())),
                               preferred_element_type=jnp.float32)
    total_blocks = cum_incl[:, E - 1:E]

    rank0 = rank_scr[pl.ds(0, T), :]
    rank1 = rank_scr[pl.ds(T, T), :]
    slot0 = jnp.sum(oh0 * (rank0 - 1.0 + pad_base), axis=1, keepdims=True)
    slot1 = jnp.sum(oh1 * (rank1 - 1.0 + pad_base), axis=1, keepdims=True)

    lane128 = lax.broadcasted_iota(jnp.int32, (T, 128), 1)
    meta = jnp.where(lane128 == 0, slot0,
                     jnp.where(lane128 == 1, slot1,
                               jnp.where(lane128 == 2, w0, w1)))
    meta_ref[...] = meta

    gidx = lax.broadcasted_iota(jnp.int32, (64, E), 0).astype(jnp.float32)
    bexp = jnp.sum((gidx >= cum_incl).astype(jnp.float32), axis=1,
                   keepdims=True)
    bexp = jnp.minimum(bexp, float(E - 1))
    row = lax.broadcasted_iota(jnp.int32, (64, 1), 0)
    vals = jnp.where(row == G, total_blocks, bexp)
    blk_ref[...] = jnp.broadcast_to(vals, (64, 128)).astype(jnp.int32)


def _dispatch_body(x_hbm, slots_hbm, xs_hbm, idx0_v, idx1_v, rows_v,
                   sem0, sem1):
    wid = lax.axis_index("s") * NC + lax.axis_index("c")
    base = wid * TOK_W

    def chunk(c, carry):
        t0 = base + c * CH
        pltpu.sync_copy(slots_hbm.at[pl.ds(t0, CH)], idx0_v)
        pltpu.sync_copy(slots_hbm.at[pl.ds(T + t0, CH)], idx1_v)
        pltpu.sync_copy(x_hbm.at[pl.ds(t0, CH)], rows_v)
        cp0 = pltpu.async_copy(rows_v, xs_hbm.at[idx0_v], sem0)
        cp1 = pltpu.async_copy(rows_v, xs_hbm.at[idx1_v], sem1)
        cp0.wait()
        cp1.wait()
        return carry

    lax.fori_loop(0, TOK_W // CH, chunk, 0)


def _ffn_body(pref_ref, xs_ref, wgu_ref, wd_ref, ys_ref):
    g = pl.program_id(0)
    nb = pref_ref[G]

    @pl.when(g < nb)
    def _():
        x = xs_ref[...]
        wgu = wgu_ref[0]
        gu = lax.dot_general(x, wgu, (((1,), (1,)), ((), ())),
                             preferred_element_type=jnp.float32)
        gate = gu[:, :F]
        up = gu[:, F:]
        h = gate * (1.0 / (1.0 + jnp.exp(-gate))) * up
        wd = wd_ref[0]
        ys_ref[...] = lax.dot_general(h, wd, (((1,), (1,)), ((), ())),
                                      preferred_element_type=jnp.float32)


def _gather_body(ys_hbm, slots_hbm, y0_hbm, y1_hbm, idx0_v, idx1_v,
                 r0_v, r1_v, sem0, sem1):
    wid = lax.axis_index("s") * NC + lax.axis_index("c")
    base = wid * TOK_W

    def chunk(c, carry):
        t0 = base + c * CH
        pltpu.sync_copy(slots_hbm.at[pl.ds(t0, CH)], idx0_v)
        pltpu.sync_copy(slots_hbm.at[pl.ds(T + t0, CH)], idx1_v)
        cp0 = pltpu.async_copy(ys_hbm.at[idx0_v], r0_v, sem0)
        cp1 = pltpu.async_copy(ys_hbm.at[idx1_v], r1_v, sem1)
        cp0.wait()
        cp1.wait()
        pltpu.sync_copy(r0_v, y0_hbm.at[pl.ds(t0, CH)])
        pltpu.sync_copy(r1_v, y1_hbm.at[pl.ds(t0, CH)])
        return carry

    lax.fori_loop(0, TOK_W // CH, chunk, 0)


def _combine_body(y0_ref, y1_ref, meta_ref, out_ref):
    w0 = meta_ref[:, 2:3]
    w1 = meta_ref[:, 3:4]
    out_ref[...] = y0_ref[...] * w0 + y1_ref[...] * w1


def kernel(hidden_states, Wg, Wgu, Wd):
    x = hidden_states

    meta, blk = pl.pallas_call(
        _router_body,
        out_shape=[
            jax.ShapeDtypeStruct((T, 128), jnp.float32),
            jax.ShapeDtypeStruct((64, 128), jnp.int32),
        ],
        scratch_shapes=[
            pltpu.VMEM((2 * T, E), jnp.float32),
            pltpu.VMEM((2 * T, E), jnp.float32),
        ],
    )(x, Wg)

    slots = jnp.concatenate([meta[:, 0], meta[:, 1]]).astype(jnp.int32)
    prefetch = blk[:G + 1, 0]

    mesh = plsc.VectorSubcoreMesh(core_axis_name="c", subcore_axis_name="s")

    xs = pl.kernel(
        _dispatch_body,
        out_type=jax.ShapeDtypeStruct((S, D), jnp.float32),
        mesh=mesh,
        scratch_types=[
            pltpu.VMEM((CH,), jnp.int32),
            pltpu.VMEM((CH,), jnp.int32),
            pltpu.VMEM((CH, D), jnp.float32),
            pltpu.SemaphoreType.DMA,
            pltpu.SemaphoreType.DMA,
        ],
    )(x, slots)

    grid_spec = pltpu.PrefetchScalarGridSpec(
        num_scalar_prefetch=1,
        grid=(G,),
        in_specs=[
            pl.BlockSpec((R, D), lambda g, pref: (g, 0)),
            pl.BlockSpec((1, 2 * F, D),
                         lambda g, pref: (pref[jnp.minimum(g, pref[G] - 1)],
                                          0, 0)),
            pl.BlockSpec((1, D, F),
                         lambda g, pref: (pref[jnp.minimum(g, pref[G] - 1)],
                                          0, 0)),
        ],
        out_specs=pl.BlockSpec((R, D), lambda g, pref: (g, 0)),
    )
    ys = pl.pallas_call(
        _ffn_body,
        grid_spec=grid_spec,
        out_shape=jax.ShapeDtypeStruct((S, D), jnp.float32),
        compiler_params=pltpu.CompilerParams(
            dimension_semantics=("arbitrary",)),
    )(prefetch, xs, Wgu, Wd)

    y0, y1 = pl.kernel(
        _gather_body,
        out_type=[
            jax.ShapeDtypeStruct((T, D), jnp.float32),
            jax.ShapeDtypeStruct((T, D), jnp.float32),
        ],
        mesh=mesh,
        scratch_types=[
            pltpu.VMEM((CH,), jnp.int32),
            pltpu.VMEM((CH,), jnp.int32),
            pltpu.VMEM((CH, D), jnp.float32),
            pltpu.VMEM((CH, D), jnp.float32),
            pltpu.SemaphoreType.DMA,
            pltpu.SemaphoreType.DMA,
        ],
    )(ys, slots)

    TB = 512
    out = pl.pallas_call(
        _combine_body,
        grid=(T // TB,),
        in_specs=[
            pl.BlockSpec((TB, D), lambda i: (i, 0)),
            pl.BlockSpec((TB, D), lambda i: (i, 0)),
            pl.BlockSpec((TB, 128), lambda i: (i, 0)),
        ],
        out_specs=pl.BlockSpec((TB, D), lambda i: (i, 0)),
        out_shape=jax.ShapeDtypeStruct((T, D), jnp.float32),
    )(y0, y1, meta)
    return out
